# initial kernel scaffold (unmeasured)
import jax
import jax.numpy as jnp
from jax import lax
from jax.experimental import pallas as pl
from jax.experimental.pallas import tpu as pltpu

N_DEV = 8


def kernel(x, w_mat, scale_x, scale_w):
    m_per, k = x.shape
    k2, n = w_mat.shape
    assert k == k2
    n_per = n // N_DEV
    f8 = jnp.float8_e4m3fn

    def body(x_ref, w_hbm, sx_ref, sw_ref, out_ref,
             x8_ref, wblk_ref, w8_ref, ybuf_ref,
             dma_sems, send_sems, recv_sems):
        my_pos = lax.axis_index("i")
        s = sx_ref[0] * sw_ref[0]

        def wdma(d):
            j = (my_pos + d) % N_DEV
            return pltpu.make_async_copy(
                w_hbm.at[:, pl.ds(j * n_per, n_per)],
                wblk_ref.at[d % 2],
                dma_sems.at[d % 2],
            )

        wdma(0).start()

        x8_ref[...] = x_ref[...].astype(f8)

        rdmas = []
        for d in range(N_DEV):
            if d < N_DEV - 1:
                wdma(d + 1).start()
            wdma(d).wait()
            w8_ref[...] = wblk_ref[d % 2].astype(f8)
            y = lax.dot_general(
                x8_ref[...], w8_ref[...],
                dimension_numbers=(((1,), (0,)), ((), ())),
                preferred_element_type=jnp.float32,
            ) * s
            if d == 0:
                out_ref[pl.ds(my_pos * m_per, m_per), :] = y
            else:
                ybuf_ref[d - 1, :, :] = y
                j = (my_pos + d) % N_DEV
                rdma = pltpu.make_async_remote_copy(
                    src_ref=ybuf_ref.at[d - 1],
                    dst_ref=out_ref.at[pl.ds(my_pos * m_per, m_per), :],
                    send_sem=send_sems.at[d - 1],
                    recv_sem=recv_sems.at[d - 1],
                    device_id=(j,),
                    device_id_type=pl.DeviceIdType.MESH,
                )
                rdma.start()
                rdmas.append(rdma)

        for rdma in rdmas:
            rdma.wait_send()

        for d in range(1, N_DEV):
            src = (my_pos - d) % N_DEV
            recv = pltpu.make_async_remote_copy(
                src_ref=ybuf_ref.at[d - 1],
                dst_ref=out_ref.at[pl.ds(src * m_per, m_per), :],
                send_sem=send_sems.at[d - 1],
                recv_sem=recv_sems.at[d - 1],
                device_id=((my_pos + d) % N_DEV,),
                device_id_type=pl.DeviceIdType.MESH,
            )
            recv.wait_recv()

    return pl.pallas_call(
        body,
        out_shape=jax.ShapeDtypeStruct((N_DEV * m_per, n_per), jnp.float32),
        in_specs=[
            pl.BlockSpec(memory_space=pltpu.VMEM),
            pl.BlockSpec(memory_space=pltpu.ANY),
            pl.BlockSpec(memory_space=pltpu.SMEM),
            pl.BlockSpec(memory_space=pltpu.SMEM),
        ],
        out_specs=pl.BlockSpec(memory_space=pltpu.VMEM),
        scratch_shapes=[
            pltpu.VMEM((m_per, k), f8),
            pltpu.VMEM((2, k, n_per), jnp.float32),
            pltpu.VMEM((k, n_per), f8),
            pltpu.VMEM((N_DEV - 1, m_per, n_per), jnp.float32),
            pltpu.SemaphoreType.DMA((2,)),
            pltpu.SemaphoreType.DMA((N_DEV - 1,)),
            pltpu.SemaphoreType.DMA((N_DEV - 1,)),
        ],
    )(x, w_mat, scale_x, scale_w)


# baseline (device time: 51380 ns/iter reference)
import jax
import jax.numpy as jnp
from jax import lax
from jax.experimental import pallas as pl
from jax.experimental.pallas import tpu as pltpu

N_DEV = 8


def kernel(x, w_mat, scale_x, scale_w):
    m_per, k = x.shape
    k2, n = w_mat.shape
    assert k == k2
    n_per = n // N_DEV
    f8 = jnp.float8_e4m3fn

    def body(x_ref, w_hbm, sx_ref, sw_ref, out_ref,
             x8_ref, wblk_ref, w8_ref, ybuf_ref,
             dma_sems, send_sems, recv_sems):
        my_pos = lax.axis_index("i")
        s = sx_ref[0] * sw_ref[0]

        def wdma(d):
            j = (my_pos + d) % N_DEV
            return pltpu.make_async_copy(
                w_hbm.at[:, pl.ds(j * n_per, n_per)],
                wblk_ref.at[d % 2],
                dma_sems.at[d % 2],
            )

        wdma(0).start()

        x8_ref[...] = x_ref[...].astype(f8)

        rdmas = []
        for d in range(N_DEV):
            if d < N_DEV - 1:
                wdma(d + 1).start()
            wdma(d).wait()
            w8_ref[...] = wblk_ref[d % 2].astype(f8)
            y = lax.dot_general(
                x8_ref[...], w8_ref[...],
                dimension_numbers=(((1,), (0,)), ((), ())),
                preferred_element_type=jnp.float32,
            ) * s
            if d == 0:
                out_ref[pl.ds(my_pos * m_per, m_per), :] = y
            else:
                ybuf_ref[d - 1, :, :] = y
                j = (my_pos + d) % N_DEV
                rdma = pltpu.make_async_remote_copy(
                    src_ref=ybuf_ref.at[d - 1],
                    dst_ref=out_ref.at[pl.ds(my_pos * m_per, m_per), :],
                    send_sem=send_sems.at[d - 1],
                    recv_sem=recv_sems.at[d - 1],
                    device_id=(j,),
                    device_id_type=pl.DeviceIdType.MESH,
                )
                rdma.start()
                rdmas.append(rdma)

        for rdma in rdmas:
            rdma.wait_send()

        for d in range(1, N_DEV):
            src = (my_pos - d) % N_DEV
            recv = pltpu.make_async_remote_copy(
                src_ref=ybuf_ref.at[d - 1],
                dst_ref=out_ref.at[pl.ds(src * m_per, m_per), :],
                send_sem=send_sems.at[d - 1],
                recv_sem=recv_sems.at[d - 1],
                device_id=((my_pos + d) % N_DEV,),
                device_id_type=pl.DeviceIdType.MESH,
            )
            recv.wait_recv()

    return pl.pallas_call(
        body,
        out_shape=jax.ShapeDtypeStruct((N_DEV * m_per, n_per), jnp.float32),
        in_specs=[
            pl.BlockSpec(memory_space=pltpu.VMEM),
            pl.BlockSpec(memory_space=pl.ANY),
            pl.BlockSpec(memory_space=pltpu.SMEM),
            pl.BlockSpec(memory_space=pltpu.SMEM),
        ],
        out_specs=pl.BlockSpec(memory_space=pltpu.VMEM),
        scratch_shapes=[
            pltpu.VMEM((m_per, k), f8),
            pltpu.VMEM((2, k, n_per), jnp.float32),
            pltpu.VMEM((k, n_per), f8),
            pltpu.VMEM((N_DEV - 1, m_per, n_per), jnp.float32),
            pltpu.SemaphoreType.DMA((2,)),
            pltpu.SemaphoreType.DMA((N_DEV - 1,)),
            pltpu.SemaphoreType.DMA((N_DEV - 1,)),
        ],
    )(x, w_mat, scale_x, scale_w)


# device time: 34717 ns/iter; 1.4800x vs baseline; 1.4800x over previous
import jax
import jax.numpy as jnp
from jax import lax
from jax.experimental import pallas as pl
from jax.experimental.pallas import tpu as pltpu

N_DEV = 8


def kernel(x, w_mat, scale_x, scale_w):
    m_per, k = x.shape
    k2, n = w_mat.shape
    assert k == k2
    n_per = n // N_DEV
    f8 = jnp.float8_e4m3fn

    def body(x_ref, w_hbm, sx_ref, sw_ref, out_ref,
             x8_ref, wblk_ref, w8_ref, ybuf_ref, rbuf_ref,
             dma_sems, send_sems, recv_sems):
        my_pos = lax.axis_index("i")
        s = sx_ref[0] * sw_ref[0]

        def wdma(d):
            j = (my_pos + d) % N_DEV
            return pltpu.make_async_copy(
                w_hbm.at[:, pl.ds(j * n_per, n_per)],
                wblk_ref.at[d % 2],
                dma_sems.at[d % 2],
            )

        wdma(0).start()

        x8_ref[...] = x_ref[...].astype(f8)

        rdmas = []
        for d in range(N_DEV):
            if d < N_DEV - 1:
                wdma(d + 1).start()
            wdma(d).wait()
            w8_ref[...] = wblk_ref[d % 2].astype(f8)
            y = lax.dot_general(
                x8_ref[...], w8_ref[...],
                dimension_numbers=(((1,), (0,)), ((), ())),
                preferred_element_type=jnp.float32,
            ) * s
            if d == 0:
                out_ref[pl.ds(my_pos * m_per, m_per), :] = y
            else:
                ybuf_ref[d - 1, :, :] = y.astype(jnp.bfloat16)
                j = (my_pos + d) % N_DEV
                rdma = pltpu.make_async_remote_copy(
                    src_ref=ybuf_ref.at[d - 1],
                    dst_ref=rbuf_ref.at[d - 1],
                    send_sem=send_sems.at[d - 1],
                    recv_sem=recv_sems.at[d - 1],
                    device_id=(j,),
                    device_id_type=pl.DeviceIdType.MESH,
                )
                rdma.start()
                rdmas.append(rdma)

        for d in range(1, N_DEV):
            src = (my_pos - d) % N_DEV
            recv = pltpu.make_async_remote_copy(
                src_ref=ybuf_ref.at[d - 1],
                dst_ref=rbuf_ref.at[d - 1],
                send_sem=send_sems.at[d - 1],
                recv_sem=recv_sems.at[d - 1],
                device_id=((my_pos + d) % N_DEV,),
                device_id_type=pl.DeviceIdType.MESH,
            )
            recv.wait_recv()
            out_ref[pl.ds(src * m_per, m_per), :] = (
                rbuf_ref[d - 1, :, :].astype(jnp.float32))

        for rdma in rdmas:
            rdma.wait_send()

    return pl.pallas_call(
        body,
        out_shape=jax.ShapeDtypeStruct((N_DEV * m_per, n_per), jnp.float32),
        in_specs=[
            pl.BlockSpec(memory_space=pltpu.VMEM),
            pl.BlockSpec(memory_space=pl.ANY),
            pl.BlockSpec(memory_space=pltpu.SMEM),
            pl.BlockSpec(memory_space=pltpu.SMEM),
        ],
        out_specs=pl.BlockSpec(memory_space=pltpu.VMEM),
        scratch_shapes=[
            pltpu.VMEM((m_per, k), f8),
            pltpu.VMEM((2, k, n_per), jnp.float32),
            pltpu.VMEM((k, n_per), f8),
            pltpu.VMEM((N_DEV - 1, m_per, n_per), jnp.bfloat16),
            pltpu.VMEM((N_DEV - 1, m_per, n_per), jnp.bfloat16),
            pltpu.SemaphoreType.DMA((2,)),
            pltpu.SemaphoreType.DMA((N_DEV - 1,)),
            pltpu.SemaphoreType.DMA((N_DEV - 1,)),
        ],
    )(x, w_mat, scale_x, scale_w)


# device time: 30691 ns/iter; 1.6741x vs baseline; 1.1312x over previous
import jax
import jax.numpy as jnp
from jax import lax
from jax.experimental import pallas as pl
from jax.experimental.pallas import tpu as pltpu

N_DEV = 8
DEPTH = 4


def kernel(x, w_mat, scale_x, scale_w):
    m_per, k = x.shape
    k2, n = w_mat.shape
    assert k == k2
    n_per = n // N_DEV
    f8 = jnp.float8_e4m3fn

    def body(x_ref, w_hbm, sx_ref, sw_ref, out_ref,
             x8_ref, wblk_ref, w8_ref, ybuf_ref, rbuf_ref,
             dma_sems, send_sems, recv_sems):
        my_pos = lax.axis_index("i")

        barrier_sem = pltpu.get_barrier_semaphore()
        for p in range(1, N_DEV):
            pl.semaphore_signal(
                barrier_sem, inc=1,
                device_id=((my_pos + p) % N_DEV,),
                device_id_type=pl.DeviceIdType.MESH,
            )
        pl.semaphore_wait(barrier_sem, N_DEV - 1)

        s = sx_ref[0] * sw_ref[0]

        def wdma(d):
            j = (my_pos + d) % N_DEV
            return pltpu.make_async_copy(
                w_hbm.at[:, pl.ds(j * n_per, n_per)],
                wblk_ref.at[d % DEPTH],
                dma_sems.at[d % DEPTH],
            )

        for d in range(DEPTH - 1):
            wdma(d).start()

        x8_ref[...] = x_ref[...].astype(f8)

        rdmas = []
        for d in range(N_DEV):
            if d + DEPTH - 1 < N_DEV:
                wdma(d + DEPTH - 1).start()
            wdma(d).wait()
            w8_ref[...] = wblk_ref[d % DEPTH].astype(f8)
            y = lax.dot_general(
                x8_ref[...], w8_ref[...],
                dimension_numbers=(((1,), (0,)), ((), ())),
                preferred_element_type=jnp.float32,
            ) * s
            if d == 0:
                out_ref[pl.ds(my_pos * m_per, m_per), :] = y
            else:
                ybuf_ref[d - 1, :, :] = y.astype(jnp.bfloat16)
                j = (my_pos + d) % N_DEV
                rdma = pltpu.make_async_remote_copy(
                    src_ref=ybuf_ref.at[d - 1],
                    dst_ref=rbuf_ref.at[d - 1],
                    send_sem=send_sems.at[d - 1],
                    recv_sem=recv_sems.at[d - 1],
                    device_id=(j,),
                    device_id_type=pl.DeviceIdType.MESH,
                )
                rdma.start()
                rdmas.append(rdma)

        for d in range(1, N_DEV):
            src = (my_pos - d) % N_DEV
            recv = pltpu.make_async_remote_copy(
                src_ref=ybuf_ref.at[d - 1],
                dst_ref=rbuf_ref.at[d - 1],
                send_sem=send_sems.at[d - 1],
                recv_sem=recv_sems.at[d - 1],
                device_id=((my_pos + d) % N_DEV,),
                device_id_type=pl.DeviceIdType.MESH,
            )
            recv.wait_recv()
            out_ref[pl.ds(src * m_per, m_per), :] = (
                rbuf_ref[d - 1, :, :].astype(jnp.float32))

        for rdma in rdmas:
            rdma.wait_send()

    return pl.pallas_call(
        body,
        out_shape=jax.ShapeDtypeStruct((N_DEV * m_per, n_per), jnp.float32),
        in_specs=[
            pl.BlockSpec(memory_space=pltpu.VMEM),
            pl.BlockSpec(memory_space=pl.ANY),
            pl.BlockSpec(memory_space=pltpu.SMEM),
            pl.BlockSpec(memory_space=pltpu.SMEM),
        ],
        out_specs=pl.BlockSpec(memory_space=pltpu.VMEM),
        scratch_shapes=[
            pltpu.VMEM((m_per, k), f8),
            pltpu.VMEM((DEPTH, k, n_per), jnp.float32),
            pltpu.VMEM((k, n_per), f8),
            pltpu.VMEM((N_DEV - 1, m_per, n_per), jnp.bfloat16),
            pltpu.VMEM((N_DEV - 1, m_per, n_per), jnp.bfloat16),
            pltpu.SemaphoreType.DMA((DEPTH,)),
            pltpu.SemaphoreType.DMA((N_DEV - 1,)),
            pltpu.SemaphoreType.DMA((N_DEV - 1,)),
        ],
        compiler_params=pltpu.CompilerParams(collective_id=0),
    )(x, w_mat, scale_x, scale_w)


# device time: 29342 ns/iter; 1.7511x vs baseline; 1.0460x over previous
import jax
import jax.numpy as jnp
from jax import lax
from jax.experimental import pallas as pl
from jax.experimental.pallas import tpu as pltpu

N_DEV = 8
DEPTH = 4


def kernel(x, w_mat, scale_x, scale_w):
    m_per, k = x.shape
    k2, n = w_mat.shape
    assert k == k2
    n_per = n // N_DEV
    f8 = jnp.float8_e4m3fn

    def body(x_ref, w_hbm, sx_ref, sw_ref, out_ref,
             x8_ref, wblk_ref, w8_ref, ybuf_ref, rbuf_ref,
             dma_sems, send_sems, recv_sems):
        my_pos = lax.axis_index("i")

        barrier_sem = pltpu.get_barrier_semaphore()
        for p in range(1, N_DEV):
            pl.semaphore_signal(
                barrier_sem, inc=1,
                device_id=((my_pos + p) % N_DEV,),
                device_id_type=pl.DeviceIdType.MESH,
            )
        pl.semaphore_wait(barrier_sem, N_DEV - 1)

        s = sx_ref[0] * sw_ref[0]

        def wdma(t):
            j = (my_pos + 1 + t) % N_DEV
            return pltpu.make_async_copy(
                w_hbm.at[:, pl.ds(j * n_per, n_per)],
                wblk_ref.at[t % DEPTH],
                dma_sems.at[t % DEPTH],
            )

        for t in range(DEPTH - 1):
            wdma(t).start()

        x8_ref[...] = x_ref[...].astype(f8)

        rdmas = []
        for t in range(N_DEV):
            d = t + 1
            if t + DEPTH - 1 < N_DEV:
                wdma(t + DEPTH - 1).start()
            wdma(t).wait()
            w8_ref[...] = wblk_ref[t % DEPTH].astype(f8)
            y = lax.dot_general(
                x8_ref[...], w8_ref[...],
                dimension_numbers=(((1,), (0,)), ((), ())),
                preferred_element_type=jnp.float32,
            ) * s
            if t == N_DEV - 1:
                out_ref[pl.ds(my_pos * m_per, m_per), :] = y
            else:
                ybuf_ref[d - 1, :, :] = y.astype(jnp.bfloat16)
                j = (my_pos + d) % N_DEV
                rdma = pltpu.make_async_remote_copy(
                    src_ref=ybuf_ref.at[d - 1],
                    dst_ref=rbuf_ref.at[d - 1],
                    send_sem=send_sems.at[d - 1],
                    recv_sem=recv_sems.at[d - 1],
                    device_id=(j,),
                    device_id_type=pl.DeviceIdType.MESH,
                )
                rdma.start()
                rdmas.append(rdma)

        for d in range(1, N_DEV):
            src = (my_pos - d) % N_DEV
            recv = pltpu.make_async_remote_copy(
                src_ref=ybuf_ref.at[d - 1],
                dst_ref=rbuf_ref.at[d - 1],
                send_sem=send_sems.at[d - 1],
                recv_sem=recv_sems.at[d - 1],
                device_id=((my_pos + d) % N_DEV,),
                device_id_type=pl.DeviceIdType.MESH,
            )
            recv.wait_recv()
            out_ref[pl.ds(src * m_per, m_per), :] = (
                rbuf_ref[d - 1, :, :].astype(jnp.float32))

        for rdma in rdmas:
            rdma.wait_send()

    return pl.pallas_call(
        body,
        out_shape=jax.ShapeDtypeStruct((N_DEV * m_per, n_per), jnp.float32),
        in_specs=[
            pl.BlockSpec(memory_space=pltpu.VMEM),
            pl.BlockSpec(memory_space=pl.ANY),
            pl.BlockSpec(memory_space=pltpu.SMEM),
            pl.BlockSpec(memory_space=pltpu.SMEM),
        ],
        out_specs=pl.BlockSpec(memory_space=pltpu.VMEM),
        scratch_shapes=[
            pltpu.VMEM((m_per, k), f8),
            pltpu.VMEM((DEPTH, k, n_per), jnp.float32),
            pltpu.VMEM((k, n_per), f8),
            pltpu.VMEM((N_DEV - 1, m_per, n_per), jnp.bfloat16),
            pltpu.VMEM((N_DEV - 1, m_per, n_per), jnp.bfloat16),
            pltpu.SemaphoreType.DMA((DEPTH,)),
            pltpu.SemaphoreType.DMA((N_DEV - 1,)),
            pltpu.SemaphoreType.DMA((N_DEV - 1,)),
        ],
        compiler_params=pltpu.CompilerParams(collective_id=0),
    )(x, w_mat, scale_x, scale_w)


# device time: 29334 ns/iter; 1.7516x vs baseline; 1.0003x over previous
import jax
import jax.numpy as jnp
from jax import lax
from jax.experimental import pallas as pl
from jax.experimental.pallas import tpu as pltpu

N_DEV = 8
DEPTH = 4


def kernel(x, w_mat, scale_x, scale_w):
    m_per, k = x.shape
    k2, n = w_mat.shape
    assert k == k2
    n_per = n // N_DEV
    f8 = jnp.float8_e4m3fn

    def body(x_ref, w_hbm, sx_ref, sw_ref, out_ref,
             x8_ref, wblk_ref, ybuf_ref, rbuf_ref,
             dma_sems, send_sems, recv_sems):
        my_pos = lax.axis_index("i")

        barrier_sem = pltpu.get_barrier_semaphore()
        for p in range(1, N_DEV):
            pl.semaphore_signal(
                barrier_sem, inc=1,
                device_id=((my_pos + p) % N_DEV,),
                device_id_type=pl.DeviceIdType.MESH,
            )
        pl.semaphore_wait(barrier_sem, N_DEV - 1)

        s = sx_ref[0] * sw_ref[0]

        def wdma(t):
            j = (my_pos + 1 + t) % N_DEV
            return pltpu.make_async_copy(
                w_hbm.at[:, pl.ds(j * n_per, n_per)],
                wblk_ref.at[t % DEPTH],
                dma_sems.at[t % DEPTH],
            )

        for t in range(DEPTH - 1):
            wdma(t).start()

        x8_ref[...] = x_ref[...].astype(f8)

        rdmas = []
        for t in range(N_DEV):
            d = t + 1
            if t + DEPTH - 1 < N_DEV:
                wdma(t + DEPTH - 1).start()
            wdma(t).wait()
            y = lax.dot_general(
                x8_ref[...], wblk_ref[t % DEPTH].astype(f8),
                dimension_numbers=(((1,), (0,)), ((), ())),
                preferred_element_type=jnp.float32,
            ) * s
            if t == N_DEV - 1:
                out_ref[pl.ds(my_pos * m_per, m_per), :] = y
            else:
                ybuf_ref[d - 1, :, :] = y.astype(jnp.bfloat16)
                j = (my_pos + d) % N_DEV
                rdma = pltpu.make_async_remote_copy(
                    src_ref=ybuf_ref.at[d - 1],
                    dst_ref=rbuf_ref.at[d - 1],
                    send_sem=send_sems.at[d - 1],
                    recv_sem=recv_sems.at[d - 1],
                    device_id=(j,),
                    device_id_type=pl.DeviceIdType.MESH,
                )
                rdma.start()
                rdmas.append(rdma)

        for d in range(1, N_DEV):
            src = (my_pos - d) % N_DEV
            recv = pltpu.make_async_remote_copy(
                src_ref=ybuf_ref.at[d - 1],
                dst_ref=rbuf_ref.at[d - 1],
                send_sem=send_sems.at[d - 1],
                recv_sem=recv_sems.at[d - 1],
                device_id=((my_pos + d) % N_DEV,),
                device_id_type=pl.DeviceIdType.MESH,
            )
            recv.wait_recv()
            out_ref[pl.ds(src * m_per, m_per), :] = (
                rbuf_ref[d - 1, :, :].astype(jnp.float32))

        for rdma in rdmas:
            rdma.wait_send()

    return pl.pallas_call(
        body,
        out_shape=jax.ShapeDtypeStruct((N_DEV * m_per, n_per), jnp.float32),
        in_specs=[
            pl.BlockSpec(memory_space=pltpu.VMEM),
            pl.BlockSpec(memory_space=pl.ANY),
            pl.BlockSpec(memory_space=pltpu.SMEM),
            pl.BlockSpec(memory_space=pltpu.SMEM),
        ],
        out_specs=pl.BlockSpec(memory_space=pltpu.VMEM),
        scratch_shapes=[
            pltpu.VMEM((m_per, k), f8),
            pltpu.VMEM((DEPTH, k, n_per), jnp.float32),
            pltpu.VMEM((N_DEV - 1, m_per, n_per), jnp.bfloat16),
            pltpu.VMEM((N_DEV - 1, m_per, n_per), jnp.bfloat16),
            pltpu.SemaphoreType.DMA((DEPTH,)),
            pltpu.SemaphoreType.DMA((N_DEV - 1,)),
            pltpu.SemaphoreType.DMA((N_DEV - 1,)),
        ],
        compiler_params=pltpu.CompilerParams(collective_id=0),
    )(x, w_mat, scale_x, scale_w)


# device time: 28891 ns/iter; 1.7784x vs baseline; 1.0153x over previous
import jax
import jax.numpy as jnp
from jax import lax
from jax.experimental import pallas as pl
from jax.experimental.pallas import tpu as pltpu

N_DEV = 8
DEPTH = 6


def kernel(x, w_mat, scale_x, scale_w):
    m_per, k = x.shape
    k2, n = w_mat.shape
    assert k == k2
    n_per = n // N_DEV
    f8 = jnp.float8_e4m3fn

    def body(x_ref, w_hbm, sx_ref, sw_ref, out_ref,
             x8_ref, wblk_ref, ybuf_ref, rbuf_ref,
             dma_sems, send_sems, recv_sems):
        my_pos = lax.axis_index("i")

        barrier_sem = pltpu.get_barrier_semaphore()
        for p in range(1, N_DEV):
            pl.semaphore_signal(
                barrier_sem, inc=1,
                device_id=((my_pos + p) % N_DEV,),
                device_id_type=pl.DeviceIdType.MESH,
            )
        pl.semaphore_wait(barrier_sem, N_DEV - 1)

        s = sx_ref[0] * sw_ref[0]

        def wdma(t):
            j = (my_pos + 1 + t) % N_DEV
            return pltpu.make_async_copy(
                w_hbm.at[:, pl.ds(j * n_per, n_per)],
                wblk_ref.at[t % DEPTH],
                dma_sems.at[t % DEPTH],
            )

        for t in range(DEPTH - 1):
            wdma(t).start()

        x8_ref[...] = x_ref[...].astype(f8)

        rdmas = []
        for t in range(N_DEV):
            d = t + 1
            if t + DEPTH - 1 < N_DEV:
                wdma(t + DEPTH - 1).start()
            wdma(t).wait()
            y = lax.dot_general(
                x8_ref[...], wblk_ref[t % DEPTH].astype(f8),
                dimension_numbers=(((1,), (0,)), ((), ())),
                preferred_element_type=jnp.float32,
            ) * s
            if t == N_DEV - 1:
                out_ref[pl.ds(my_pos * m_per, m_per), :] = y
            else:
                ybuf_ref[d - 1, :, :] = y.astype(jnp.bfloat16)
                j = (my_pos + d) % N_DEV
                rdma = pltpu.make_async_remote_copy(
                    src_ref=ybuf_ref.at[d - 1],
                    dst_ref=rbuf_ref.at[d - 1],
                    send_sem=send_sems.at[d - 1],
                    recv_sem=recv_sems.at[d - 1],
                    device_id=(j,),
                    device_id_type=pl.DeviceIdType.MESH,
                )
                rdma.start()
                rdmas.append(rdma)

        for d in range(1, N_DEV):
            src = (my_pos - d) % N_DEV
            recv = pltpu.make_async_remote_copy(
                src_ref=ybuf_ref.at[d - 1],
                dst_ref=rbuf_ref.at[d - 1],
                send_sem=send_sems.at[d - 1],
                recv_sem=recv_sems.at[d - 1],
                device_id=((my_pos + d) % N_DEV,),
                device_id_type=pl.DeviceIdType.MESH,
            )
            recv.wait_recv()
            out_ref[pl.ds(src * m_per, m_per), :] = (
                rbuf_ref[d - 1, :, :].astype(jnp.float32))

        for rdma in rdmas:
            rdma.wait_send()

    return pl.pallas_call(
        body,
        out_shape=jax.ShapeDtypeStruct((N_DEV * m_per, n_per), jnp.float32),
        in_specs=[
            pl.BlockSpec(memory_space=pltpu.VMEM),
            pl.BlockSpec(memory_space=pl.ANY),
            pl.BlockSpec(memory_space=pltpu.SMEM),
            pl.BlockSpec(memory_space=pltpu.SMEM),
        ],
        out_specs=pl.BlockSpec(memory_space=pltpu.VMEM),
        scratch_shapes=[
            pltpu.VMEM((m_per, k), f8),
            pltpu.VMEM((DEPTH, k, n_per), jnp.float32),
            pltpu.VMEM((N_DEV - 1, m_per, n_per), jnp.bfloat16),
            pltpu.VMEM((N_DEV - 1, m_per, n_per), jnp.bfloat16),
            pltpu.SemaphoreType.DMA((DEPTH,)),
            pltpu.SemaphoreType.DMA((N_DEV - 1,)),
            pltpu.SemaphoreType.DMA((N_DEV - 1,)),
        ],
        compiler_params=pltpu.CompilerParams(collective_id=0),
    )(x, w_mat, scale_x, scale_w)
